# baseline (speedup 1.0000x reference)
import functools

import jax
import jax.numpy as jnp
from jax import lax
from jax.experimental import pallas as pl
from jax.experimental.pallas import tpu as pltpu
from jax.experimental.pallas import tpu_sc as plsc

EMB = 64
_BN = 2000
_BE = 3200
_NC, _NS = 2, 16
_GCH = 40
_SCH = 80


def _lnorm(x, g, b):
    f = x.shape[-1]
    ones = jnp.ones((f, 1), jnp.float32)
    inv = jnp.float32(1.0 / f)
    m = jnp.dot(x, ones, preferred_element_type=jnp.float32) * inv
    d = x - m
    v = jnp.dot(d * d, ones, preferred_element_type=jnp.float32) * inv
    return d * lax.rsqrt(v + 1e-5) * g + b


def _full(shape):
    n = len(shape)
    return pl.BlockSpec(shape, lambda i, _n=n: (jnp.int32(0),) * _n)


def _rows(blk, width):
    return pl.BlockSpec((blk, width), lambda i: (i, jnp.int32(0)))


def _node_mlp(x, lng, lnb, w1t, b1, w2t, b2, projs):
    n, f = x.shape
    nproj = len(projs)

    def body(x_ref, lng_ref, lnb_ref, w1t_ref, b1_ref, w2t_ref, b2_ref, *rest):
        pw = rest[:2 * nproj]
        outs = rest[2 * nproj:]
        h = _lnorm(x_ref[...], lng_ref[...], lnb_ref[...])
        h = jnp.maximum(
            jnp.dot(h, w1t_ref[...], preferred_element_type=jnp.float32)
            + b1_ref[...], 0.0)
        h = jnp.maximum(
            jnp.dot(h, w2t_ref[...], preferred_element_type=jnp.float32)
            + b2_ref[...], 0.0)
        outs[0][...] = h
        for k in range(nproj):
            outs[1 + k][...] = (
                jnp.dot(h, pw[2 * k][...], preferred_element_type=jnp.float32)
                + pw[2 * k + 1][...])

    in_specs = [_rows(_BN, f), _full((1, f)), _full((1, f)),
                _full((f, EMB)), _full((1, EMB)),
                _full((EMB, EMB)), _full((1, EMB))]
    args = [x, lng.reshape(1, f), lnb.reshape(1, f),
            w1t, b1.reshape(1, EMB), w2t, b2.reshape(1, EMB)]
    for wt, bb in projs:
        in_specs += [_full((EMB, EMB)), _full((1, EMB))]
        args += [wt, bb.reshape(1, EMB)]
    return pl.pallas_call(
        body,
        grid=(n // _BN,),
        in_specs=in_specs,
        out_specs=[_rows(_BN, EMB)] * (1 + nproj),
        out_shape=[jax.ShapeDtypeStruct((n, EMB), jnp.float32)] * (1 + nproj),
    )(*args)


def _msg(jl, jr, fing, finb, fwt, fwb):
    e = jl.shape[0]

    def body(jl_ref, jr_ref, g_ref, b_ref, wt_ref, wb_ref, out_ref):
        x = jl_ref[...] + jr_ref[...]
        h = jnp.maximum(_lnorm(x, g_ref[...], b_ref[...]), 0.0)
        y = (jnp.dot(h, wt_ref[...], preferred_element_type=jnp.float32)
             + wb_ref[...])
        out_ref[...] = jnp.stack([y[:, :32], y[:, 32:]])

    return pl.pallas_call(
        body,
        grid=(e // _BE,),
        in_specs=[_rows(_BE, EMB), _rows(_BE, EMB),
                  _full((1, EMB)), _full((1, EMB)),
                  _full((EMB, EMB)), _full((1, EMB))],
        out_specs=pl.BlockSpec((2, _BE, 32), lambda i: (jnp.int32(0), i, jnp.int32(0))),
        out_shape=jax.ShapeDtypeStruct((2, e, 32), jnp.float32),
    )(jl, jr, fing.reshape(1, EMB), finb.reshape(1, EMB), fwt,
      fwb.reshape(1, EMB))


def _post(agg, right, pg, pb, w1t, b1, w2t, b2, projs, head):
    n = agg.shape[0]
    nproj = len(projs)

    def body(agg_ref, right_ref, pg_ref, pb_ref, w1t_ref, b1_ref,
             w2t_ref, b2_ref, *rest):
        pw = rest[:2 * nproj]
        if head is not None:
            hw1, hb1, hw2 = rest[2 * nproj:2 * nproj + 3]
            outs = rest[2 * nproj + 3:]
        else:
            outs = rest[2 * nproj:]
        hn = _lnorm(agg_ref[...], pg_ref[...], pb_ref[...])
        hcat = jnp.concatenate([hn, right_ref[...]], axis=-1)
        t = jnp.maximum(
            jnp.dot(hcat, w1t_ref[...], preferred_element_type=jnp.float32)
            + b1_ref[...], 0.0)
        emb = (jnp.dot(t, w2t_ref[...], preferred_element_type=jnp.float32)
               + b2_ref[...])
        if head is not None:
            z = jnp.maximum(
                jnp.dot(emb, hw1[...], preferred_element_type=jnp.float32)
                + hb1[...], 0.0)
            outs[0][...] = jnp.dot(z, hw2[...],
                                   preferred_element_type=jnp.float32)
        else:
            outs[0][...] = emb
            for k in range(nproj):
                outs[1 + k][...] = (
                    jnp.dot(emb, pw[2 * k][...],
                            preferred_element_type=jnp.float32)
                    + pw[2 * k + 1][...])

    in_specs = [_rows(_BN, EMB), _rows(_BN, EMB),
                _full((1, EMB)), _full((1, EMB)),
                _full((2 * EMB, EMB)), _full((1, EMB)),
                _full((EMB, EMB)), _full((1, EMB))]
    args = [agg, right, pg.reshape(1, EMB), pb.reshape(1, EMB),
            w1t, b1.reshape(1, EMB), w2t, b2.reshape(1, EMB)]
    for wt, bb in projs:
        in_specs += [_full((EMB, EMB)), _full((1, EMB))]
        args += [wt, bb.reshape(1, EMB)]
    if head is not None:
        hw1t, hb1, hw2t = head
        in_specs += [_full((EMB, EMB)), _full((1, EMB)), _full((EMB, 1))]
        args += [hw1t, hb1.reshape(1, EMB), hw2t]
        out_specs = _rows(_BN, 1)
        out_shape = jax.ShapeDtypeStruct((n, 1), jnp.float32)
    else:
        out_specs = [_rows(_BN, EMB)] * (1 + nproj)
        out_shape = [jax.ShapeDtypeStruct((n, EMB), jnp.float32)] * (1 + nproj)
    return pl.pallas_call(
        body,
        grid=(n // _BN,),
        in_specs=in_specs,
        out_specs=out_specs,
        out_shape=out_shape,
    )(*args)


def _gather(a_tab, b_tab, src, dst):
    e = src.shape[0]
    nw = _NC * _NS
    per_w = e // nw
    iters = per_w // _GCH
    mesh = plsc.VectorSubcoreMesh(core_axis_name="c", subcore_axis_name="s")

    @functools.partial(
        pl.kernel,
        out_type=(jax.ShapeDtypeStruct((e, EMB), jnp.float32),
                  jax.ShapeDtypeStruct((e, EMB), jnp.float32)),
        mesh=mesh,
        scratch_types=[
            pltpu.VMEM((per_w,), jnp.int32),
            pltpu.VMEM((per_w,), jnp.int32),
            pltpu.VMEM((2, _GCH, EMB), jnp.float32),
            pltpu.VMEM((2, _GCH, EMB), jnp.float32),
            pltpu.SemaphoreType.DMA,
            pltpu.SemaphoreType.DMA,
            pltpu.SemaphoreType.DMA,
            pltpu.SemaphoreType.DMA,
        ],
        compiler_params=pltpu.CompilerParams(use_tc_tiling_on_sc=False),
    )
    def k(a_hbm, b_hbm, src_hbm, dst_hbm, jl_hbm, jr_hbm,
          sidx, didx, bufa, bufb, sa0, sa1, sb0, sb1):
        wid = lax.axis_index("s") * _NC + lax.axis_index("c")
        pltpu.sync_copy(src_hbm.at[pl.ds(wid * per_w, per_w)], sidx)
        pltpu.sync_copy(dst_hbm.at[pl.ds(wid * per_w, per_w)], didx)
        sa = (sa0, sa1)
        sb = (sb0, sb1)

        def issue(j, b):
            @pl.when(j < iters)
            def _():
                pltpu.async_copy(a_hbm.at[sidx.at[pl.ds(j * _GCH, _GCH)]],
                                 bufa.at[jnp.int32(b)], sa[b])
                pltpu.async_copy(b_hbm.at[didx.at[pl.ds(j * _GCH, _GCH)]],
                                 bufb.at[jnp.int32(b)], sb[b])

        def step(i, b):
            pltpu.make_async_copy(a_hbm.at[sidx.at[pl.ds(i * _GCH, _GCH)]],
                                  bufa.at[jnp.int32(b)], sa[b]).wait()
            pltpu.make_async_copy(b_hbm.at[didx.at[pl.ds(i * _GCH, _GCH)]],
                                  bufb.at[jnp.int32(b)], sb[b]).wait()
            ebase = wid * per_w + i * _GCH
            pltpu.sync_copy(bufa.at[jnp.int32(b)], jl_hbm.at[pl.ds(ebase, _GCH)])
            pltpu.sync_copy(bufb.at[jnp.int32(b)], jr_hbm.at[pl.ds(ebase, _GCH)])
            issue(i + 2, b)

        issue(jnp.int32(0), 0)
        issue(jnp.int32(1), 1)

        def pair(kk, carry):
            step(2 * kk, 0)
            step(2 * kk + 1, 1)
            return carry

        lax.fori_loop(jnp.int32(0), jnp.int32(iters // 2), pair,
                      jnp.int32(0))
        for t in range((iters // 2) * 2, iters):
            step(jnp.int32(t), t % 2)

    return k(a_tab, b_tab, src, dst)


_NHALF = 25000
_AGG_ROWS = 25088
_AGG_STRIPE = _AGG_ROWS // _NS


def _scatter(msg2, dst):
    e = dst.shape[0]
    per_t = e // _NS
    iters = per_t // _SCH
    mesh = plsc.VectorSubcoreMesh(core_axis_name="c", subcore_axis_name="s")

    @functools.partial(
        pl.kernel,
        out_type=jax.ShapeDtypeStruct((_NC, 2, _AGG_ROWS, 32), jnp.float32),
        mesh=mesh,
        scratch_types=[
            pltpu.VMEM((per_t,), jnp.int32),
            pltpu.VMEM((4, _SCH), jnp.int32),
            pltpu.VMEM((4, _SCH, 32), jnp.float32),
            pltpu.VMEM((224, 32), jnp.float32),
            pltpu.VMEM_SHARED((_AGG_ROWS, 32), jnp.float32),
            pltpu.SemaphoreType.DMA,
            pltpu.SemaphoreType.DMA,
            pltpu.SemaphoreType.DMA,
            pltpu.SemaphoreType.DMA,
            pltpu.SemaphoreType.DMA,
            pltpu.SemaphoreType.DMA,
            pltpu.SemaphoreType.DMA,
            pltpu.SemaphoreType.DMA,
        ],
        compiler_params=pltpu.CompilerParams(use_tc_tiling_on_sc=False),
    )
    def k(msg_hbm, dst_hbm, out_hbm, didx, lidx, mbuf, zbuf, table,
          sm0, sm1, sm2, sm3, sa0, sa1, sa2, sa3):
        c = lax.axis_index("c")
        s = lax.axis_index("s")
        sm = (sm0, sm1, sm2, sm3)
        sad = (sa0, sa1, sa2, sa3)

        def zb(r, carry):
            zbuf[r, pl.ds(0, 16)] = jnp.zeros((16,), jnp.float32)
            zbuf[r, pl.ds(16, 16)] = jnp.zeros((16,), jnp.float32)
            return carry

        lax.fori_loop(jnp.int32(0), jnp.int32(224), zb, jnp.int32(0))
        pltpu.sync_copy(dst_hbm.at[pl.ds(s * per_t, per_t)], didx)

        for p in range(2):
            def zs(j, carry):
                pltpu.sync_copy(
                    zbuf, table.at[pl.ds(s * _AGG_STRIPE + j * 224, 224)])
                return carry

            lax.fori_loop(jnp.int32(0), jnp.int32(_AGG_STRIPE // 224), zs,
                          jnp.int32(0))
            plsc.subcore_barrier()

            lo = jnp.int32(p * _NHALF)

            def rd(j, b):
                @pl.when(j < iters)
                def _():
                    base = s * per_t + j * _SCH
                    pltpu.async_copy(msg_hbm.at[c, pl.ds(base, _SCH)],
                                     mbuf.at[jnp.int32(b)], sm[b])

            def step(i, b, wait_add):
                base = s * per_t + i * _SCH
                pltpu.make_async_copy(msg_hbm.at[c, pl.ds(base, _SCH)],
                                      mbuf.at[jnp.int32(b)], sm[b]).wait()
                for j in range(_SCH // 16):
                    v = didx[pl.ds(i * _SCH + 16 * j, 16)] - lo
                    v = jnp.where((v >= 0) & (v < _NHALF), v,
                                  jnp.int32(_NHALF))
                    lidx[jnp.int32(b), pl.ds(16 * j, 16)] = v
                pltpu.async_copy(mbuf.at[jnp.int32(b)],
                                 table.at[lidx.at[jnp.int32(b)]],
                                 sad[b], add=True)
                bn = (b + 2) % 4
                if wait_add:
                    pltpu.make_async_copy(
                        mbuf.at[jnp.int32(bn)],
                        table.at[lidx.at[jnp.int32(bn)]], sad[bn]).wait()
                rd(i + 2, bn)

            rd(jnp.int32(0), 0)
            rd(jnp.int32(1), 1)
            step(jnp.int32(0), 0, False)
            step(jnp.int32(1), 1, False)

            def quad(kk, carry):
                i = 2 + 4 * kk
                step(i, 2, True)
                step(i + 1, 3, True)
                step(i + 2, 0, True)
                step(i + 3, 1, True)
                return carry

            nquad = (iters - 2) // 4
            lax.fori_loop(jnp.int32(0), jnp.int32(nquad), quad, jnp.int32(0))
            for t in range(2 + 4 * nquad, iters):
                step(jnp.int32(t), t % 4, True)
            for t in range(max(iters - 2, 0), iters):
                bl = t % 4
                pltpu.make_async_copy(
                    mbuf.at[jnp.int32(bl)],
                    table.at[lidx.at[jnp.int32(bl)]], sad[bl]).wait()
            plsc.subcore_barrier()

            pltpu.sync_copy(
                table.at[pl.ds(s * _AGG_STRIPE, _AGG_STRIPE)],
                out_hbm.at[c, jnp.int32(p), pl.ds(s * _AGG_STRIPE, _AGG_STRIPE)])

    return k(msg2, dst)


def _assemble_agg(aggp, n):
    rows = jnp.concatenate([aggp[:, 0, :_NHALF], aggp[:, 1, :n - _NHALF]],
                           axis=1)
    return jnp.concatenate([rows[0], rows[1]], axis=-1)


def kernel(constraint_features, edge_indices, edge_features, variable_features,
           params):
    del edge_features
    p = params
    p1 = p['conv_v_to_c']
    p2 = p['conv_c_to_v']
    n = constraint_features.shape[0]
    ei = edge_indices.astype(jnp.int32)
    zb = jnp.zeros((EMB,), jnp.float32)

    b0 = p['edge_ln_b'][0]
    bias1 = p1['left_b'] + b0 * p1['edge_W'][:, 0]
    bias2 = p2['left_b'] + b0 * p2['edge_W'][:, 0]

    cons0, b1_tab = _node_mlp(
        constraint_features, p['cons_ln_g'], p['cons_ln_b'],
        p['cons_W1'].T, p['cons_b1'], p['cons_W2'].T, p['cons_b2'],
        [(p1['right_W'].T, zb)])
    var0, a1_tab, b2_tab = _node_mlp(
        variable_features, p['var_ln_g'], p['var_ln_b'],
        p['var_W1'].T, p['var_b1'], p['var_W2'].T, p['var_b2'],
        [(p1['left_W'].T, bias1), (p2['right_W'].T, zb)])

    jl1, jr1 = _gather(a1_tab, b1_tab, ei[1], ei[0])
    msg1 = _msg(jl1, jr1, p1['fin_g'], p1['fin_b'], p1['fin_W'].T,
                p1['fin_Wb'])
    agg1p = _scatter(msg1, ei[0])
    agg1 = _assemble_agg(agg1p, n)
    cons1, a2_tab = _post(
        agg1, cons0, p1['post_g'], p1['post_b'], p1['out1_W'].T,
        p1['out1_b'], p1['out2_W'].T, p1['out2_b'],
        [(p2['left_W'].T, bias2)], head=None)

    jl2, jr2 = _gather(a2_tab, b2_tab, ei[0], ei[1])
    msg2 = _msg(jl2, jr2, p2['fin_g'], p2['fin_b'], p2['fin_W'].T,
                p2['fin_Wb'])
    agg2p = _scatter(msg2, ei[1])
    agg2 = _assemble_agg(agg2p, n)
    out = _post(
        agg2, var0, p2['post_g'], p2['post_b'], p2['out1_W'].T,
        p2['out1_b'], p2['out2_W'].T, p2['out2_b'], [],
        head=(p['out_W1'].T, p['out_b1'], p['out_W2'].T))
    return out[:, 0]


# speedup vs baseline: 1.2416x; 1.2416x over previous
import functools

import jax
import jax.numpy as jnp
from jax import lax
from jax.experimental import pallas as pl
from jax.experimental.pallas import tpu as pltpu
from jax.experimental.pallas import tpu_sc as plsc

EMB = 64
_BN = 2000
_BE = 3200
_NC, _NS = 2, 16
_GCH = 40
_SCH = 80


def _lnorm(x, g, b):
    f = x.shape[-1]
    ones = jnp.ones((f, 1), jnp.float32)
    inv = jnp.float32(1.0 / f)
    m = jnp.dot(x, ones, preferred_element_type=jnp.float32) * inv
    d = x - m
    v = jnp.dot(d * d, ones, preferred_element_type=jnp.float32) * inv
    return d * lax.rsqrt(v + 1e-5) * g + b


def _full(shape):
    n = len(shape)
    return pl.BlockSpec(shape, lambda i, _n=n: (jnp.int32(0),) * _n)


def _rows(blk, width):
    return pl.BlockSpec((blk, width), lambda i: (i, jnp.int32(0)))


def _node_mlp(x, lng, lnb, w1t, b1, w2t, b2, projs):
    n, f = x.shape
    nproj = len(projs)

    def body(x_ref, lng_ref, lnb_ref, w1t_ref, b1_ref, w2t_ref, b2_ref, *rest):
        pw = rest[:2 * nproj]
        outs = rest[2 * nproj:]
        h = _lnorm(x_ref[...], lng_ref[...], lnb_ref[...])
        h = jnp.maximum(
            jnp.dot(h, w1t_ref[...], preferred_element_type=jnp.float32)
            + b1_ref[...], 0.0)
        h = jnp.maximum(
            jnp.dot(h, w2t_ref[...], preferred_element_type=jnp.float32)
            + b2_ref[...], 0.0)
        outs[0][...] = h
        for k in range(nproj):
            outs[1 + k][...] = (
                jnp.dot(h, pw[2 * k][...], preferred_element_type=jnp.float32)
                + pw[2 * k + 1][...])

    in_specs = [_rows(_BN, f), _full((1, f)), _full((1, f)),
                _full((f, EMB)), _full((1, EMB)),
                _full((EMB, EMB)), _full((1, EMB))]
    args = [x, lng.reshape(1, f), lnb.reshape(1, f),
            w1t, b1.reshape(1, EMB), w2t, b2.reshape(1, EMB)]
    for wt, bb in projs:
        in_specs += [_full((EMB, EMB)), _full((1, EMB))]
        args += [wt, bb.reshape(1, EMB)]
    return pl.pallas_call(
        body,
        grid=(n // _BN,),
        in_specs=in_specs,
        out_specs=[_rows(_BN, EMB)] * (1 + nproj),
        out_shape=[jax.ShapeDtypeStruct((n, EMB), jnp.float32)] * (1 + nproj),
    )(*args)


def _msg(j2, fing, finb, fwt, fwb):
    e = j2.shape[0]

    def body(j_ref, g_ref, b_ref, wt_ref, wb_ref, out_ref):
        j = j_ref[...]
        x = j[:, :EMB] + j[:, EMB:]
        h = jnp.maximum(_lnorm(x, g_ref[...], b_ref[...]), 0.0)
        y = (jnp.dot(h, wt_ref[...], preferred_element_type=jnp.float32)
             + wb_ref[...])
        out_ref[...] = jnp.stack([y[:, :32], y[:, 32:]])

    return pl.pallas_call(
        body,
        grid=(e // _BE,),
        in_specs=[_rows(_BE, 2 * EMB),
                  _full((1, EMB)), _full((1, EMB)),
                  _full((EMB, EMB)), _full((1, EMB))],
        out_specs=pl.BlockSpec((2, _BE, 32), lambda i: (jnp.int32(0), i, jnp.int32(0))),
        out_shape=jax.ShapeDtypeStruct((2, e, 32), jnp.float32),
    )(j2, fing.reshape(1, EMB), finb.reshape(1, EMB), fwt,
      fwb.reshape(1, EMB))


def _post(agg, right, pg, pb, w1t, b1, w2t, b2, projs, head):
    n = agg.shape[0]
    nproj = len(projs)

    def body(agg_ref, right_ref, pg_ref, pb_ref, w1t_ref, b1_ref,
             w2t_ref, b2_ref, *rest):
        pw = rest[:2 * nproj]
        if head is not None:
            hw1, hb1, hw2 = rest[2 * nproj:2 * nproj + 3]
            outs = rest[2 * nproj + 3:]
        else:
            outs = rest[2 * nproj:]
        hn = _lnorm(agg_ref[...], pg_ref[...], pb_ref[...])
        hcat = jnp.concatenate([hn, right_ref[...]], axis=-1)
        t = jnp.maximum(
            jnp.dot(hcat, w1t_ref[...], preferred_element_type=jnp.float32)
            + b1_ref[...], 0.0)
        emb = (jnp.dot(t, w2t_ref[...], preferred_element_type=jnp.float32)
               + b2_ref[...])
        if head is not None:
            z = jnp.maximum(
                jnp.dot(emb, hw1[...], preferred_element_type=jnp.float32)
                + hb1[...], 0.0)
            outs[0][...] = jnp.dot(z, hw2[...],
                                   preferred_element_type=jnp.float32)
        else:
            outs[0][...] = emb
            for k in range(nproj):
                outs[1 + k][...] = (
                    jnp.dot(emb, pw[2 * k][...],
                            preferred_element_type=jnp.float32)
                    + pw[2 * k + 1][...])

    in_specs = [_rows(_BN, EMB), _rows(_BN, EMB),
                _full((1, EMB)), _full((1, EMB)),
                _full((2 * EMB, EMB)), _full((1, EMB)),
                _full((EMB, EMB)), _full((1, EMB))]
    args = [agg, right, pg.reshape(1, EMB), pb.reshape(1, EMB),
            w1t, b1.reshape(1, EMB), w2t, b2.reshape(1, EMB)]
    for wt, bb in projs:
        in_specs += [_full((EMB, EMB)), _full((1, EMB))]
        args += [wt, bb.reshape(1, EMB)]
    if head is not None:
        hw1t, hb1, hw2t = head
        in_specs += [_full((EMB, EMB)), _full((1, EMB)), _full((EMB, 1))]
        args += [hw1t, hb1.reshape(1, EMB), hw2t]
        out_specs = _rows(_BN, 1)
        out_shape = jax.ShapeDtypeStruct((n, 1), jnp.float32)
    else:
        out_specs = [_rows(_BN, EMB)] * (1 + nproj)
        out_shape = [jax.ShapeDtypeStruct((n, EMB), jnp.float32)] * (1 + nproj)
    return pl.pallas_call(
        body,
        grid=(n // _BN,),
        in_specs=in_specs,
        out_specs=out_specs,
        out_shape=out_shape,
    )(*args)


def _gather(a_tab, b_tab, src, dst):
    e = src.shape[0]
    nw = _NC * _NS
    per_w = e // nw
    iters = per_w // _GCH
    mesh = plsc.VectorSubcoreMesh(core_axis_name="c", subcore_axis_name="s")

    @functools.partial(
        pl.kernel,
        out_type=jax.ShapeDtypeStruct((e, 2 * EMB), jnp.float32),
        mesh=mesh,
        scratch_types=[
            pltpu.VMEM((per_w,), jnp.int32),
            pltpu.VMEM((per_w,), jnp.int32),
            pltpu.VMEM((2, _GCH, EMB), jnp.float32),
            pltpu.VMEM((2, _GCH, EMB), jnp.float32),
            pltpu.SemaphoreType.DMA,
            pltpu.SemaphoreType.DMA,
            pltpu.SemaphoreType.DMA,
            pltpu.SemaphoreType.DMA,
        ],
        compiler_params=pltpu.CompilerParams(use_tc_tiling_on_sc=False),
    )
    def k(a_hbm, b_hbm, src_hbm, dst_hbm, j2_hbm,
          sidx, didx, bufa, bufb, sa0, sa1, sb0, sb1):
        wid = lax.axis_index("s") * _NC + lax.axis_index("c")
        pltpu.sync_copy(src_hbm.at[pl.ds(wid * per_w, per_w)], sidx)
        pltpu.sync_copy(dst_hbm.at[pl.ds(wid * per_w, per_w)], didx)
        sa = (sa0, sa1)
        sb = (sb0, sb1)

        def issue(j, b):
            @pl.when(j < iters)
            def _():
                pltpu.async_copy(a_hbm.at[sidx.at[pl.ds(j * _GCH, _GCH)]],
                                 bufa.at[jnp.int32(b)], sa[b])
                pltpu.async_copy(b_hbm.at[didx.at[pl.ds(j * _GCH, _GCH)]],
                                 bufb.at[jnp.int32(b)], sb[b])

        def step(i, b):
            pltpu.make_async_copy(a_hbm.at[sidx.at[pl.ds(i * _GCH, _GCH)]],
                                  bufa.at[jnp.int32(b)], sa[b]).wait()
            pltpu.make_async_copy(b_hbm.at[didx.at[pl.ds(i * _GCH, _GCH)]],
                                  bufb.at[jnp.int32(b)], sb[b]).wait()
            ebase = wid * per_w + i * _GCH
            pltpu.sync_copy(bufa.at[jnp.int32(b)],
                            j2_hbm.at[pl.ds(ebase, _GCH), pl.ds(0, EMB)])
            pltpu.sync_copy(bufb.at[jnp.int32(b)],
                            j2_hbm.at[pl.ds(ebase, _GCH), pl.ds(EMB, EMB)])
            issue(i + 2, b)

        issue(jnp.int32(0), 0)
        issue(jnp.int32(1), 1)

        def pair(kk, carry):
            step(2 * kk, 0)
            step(2 * kk + 1, 1)
            return carry

        lax.fori_loop(jnp.int32(0), jnp.int32(iters // 2), pair,
                      jnp.int32(0))
        for t in range((iters // 2) * 2, iters):
            step(jnp.int32(t), t % 2)

    return k(a_tab, b_tab, src, dst)


_NHALF = 25000
_AGG_ROWS = 25088
_AGG_STRIPE = _AGG_ROWS // _NS


def _scatter(msg2, dst):
    e = dst.shape[0]
    per_t = e // _NS
    iters = per_t // _SCH
    mesh = plsc.VectorSubcoreMesh(core_axis_name="c", subcore_axis_name="s")

    @functools.partial(
        pl.kernel,
        out_type=jax.ShapeDtypeStruct((_NC, 2, _AGG_ROWS, 32), jnp.float32),
        mesh=mesh,
        scratch_types=[
            pltpu.VMEM((per_t,), jnp.int32),
            pltpu.VMEM((4, _SCH), jnp.int32),
            pltpu.VMEM((4, _SCH, 32), jnp.float32),
            pltpu.VMEM((224, 32), jnp.float32),
            pltpu.VMEM_SHARED((_AGG_ROWS, 32), jnp.float32),
            pltpu.SemaphoreType.DMA,
            pltpu.SemaphoreType.DMA,
            pltpu.SemaphoreType.DMA,
            pltpu.SemaphoreType.DMA,
            pltpu.SemaphoreType.DMA,
            pltpu.SemaphoreType.DMA,
            pltpu.SemaphoreType.DMA,
            pltpu.SemaphoreType.DMA,
        ],
        compiler_params=pltpu.CompilerParams(use_tc_tiling_on_sc=False),
    )
    def k(msg_hbm, dst_hbm, out_hbm, didx, lidx, mbuf, zbuf, table,
          sm0, sm1, sm2, sm3, sa0, sa1, sa2, sa3):
        c = lax.axis_index("c")
        s = lax.axis_index("s")
        sm = (sm0, sm1, sm2, sm3)
        sad = (sa0, sa1, sa2, sa3)

        def zb(r, carry):
            zbuf[r, pl.ds(0, 16)] = jnp.zeros((16,), jnp.float32)
            zbuf[r, pl.ds(16, 16)] = jnp.zeros((16,), jnp.float32)
            return carry

        lax.fori_loop(jnp.int32(0), jnp.int32(224), zb, jnp.int32(0))
        pltpu.sync_copy(dst_hbm.at[pl.ds(s * per_t, per_t)], didx)

        for p in range(2):
            def zs(j, carry):
                pltpu.sync_copy(
                    zbuf, table.at[pl.ds(s * _AGG_STRIPE + j * 224, 224)])
                return carry

            lax.fori_loop(jnp.int32(0), jnp.int32(_AGG_STRIPE // 224), zs,
                          jnp.int32(0))
            plsc.subcore_barrier()

            lo = jnp.int32(p * _NHALF)

            def rd(j, b):
                @pl.when(j < iters)
                def _():
                    base = s * per_t + j * _SCH
                    pltpu.async_copy(msg_hbm.at[c, pl.ds(base, _SCH)],
                                     mbuf.at[jnp.int32(b)], sm[b])

            def step(i, b, wait_add):
                base = s * per_t + i * _SCH
                pltpu.make_async_copy(msg_hbm.at[c, pl.ds(base, _SCH)],
                                      mbuf.at[jnp.int32(b)], sm[b]).wait()
                for j in range(_SCH // 16):
                    v = didx[pl.ds(i * _SCH + 16 * j, 16)] - lo
                    v = jnp.where((v >= 0) & (v < _NHALF), v,
                                  jnp.int32(_NHALF))
                    lidx[jnp.int32(b), pl.ds(16 * j, 16)] = v
                pltpu.async_copy(mbuf.at[jnp.int32(b)],
                                 table.at[lidx.at[jnp.int32(b)]],
                                 sad[b], add=True)
                bn = (b + 2) % 4
                if wait_add:
                    pltpu.make_async_copy(
                        mbuf.at[jnp.int32(bn)],
                        table.at[lidx.at[jnp.int32(bn)]], sad[bn]).wait()
                rd(i + 2, bn)

            rd(jnp.int32(0), 0)
            rd(jnp.int32(1), 1)
            step(jnp.int32(0), 0, False)
            step(jnp.int32(1), 1, False)

            def quad(kk, carry):
                i = 2 + 4 * kk
                step(i, 2, True)
                step(i + 1, 3, True)
                step(i + 2, 0, True)
                step(i + 3, 1, True)
                return carry

            nquad = (iters - 2) // 4
            lax.fori_loop(jnp.int32(0), jnp.int32(nquad), quad, jnp.int32(0))
            for t in range(2 + 4 * nquad, iters):
                step(jnp.int32(t), t % 4, True)
            for t in range(max(iters - 2, 0), iters):
                bl = t % 4
                pltpu.make_async_copy(
                    mbuf.at[jnp.int32(bl)],
                    table.at[lidx.at[jnp.int32(bl)]], sad[bl]).wait()
            plsc.subcore_barrier()

            pltpu.sync_copy(
                table.at[pl.ds(s * _AGG_STRIPE, _AGG_STRIPE)],
                out_hbm.at[c, jnp.int32(p), pl.ds(s * _AGG_STRIPE, _AGG_STRIPE)])

    return k(msg2, dst)


def _assemble_agg(aggp, n):
    rows = jnp.concatenate([aggp[:, 0, :_NHALF], aggp[:, 1, :n - _NHALF]],
                           axis=1)
    return jnp.concatenate([rows[0], rows[1]], axis=-1)


def kernel(constraint_features, edge_indices, edge_features, variable_features,
           params):
    del edge_features
    p = params
    p1 = p['conv_v_to_c']
    p2 = p['conv_c_to_v']
    n = constraint_features.shape[0]
    ei = edge_indices.astype(jnp.int32)
    zb = jnp.zeros((EMB,), jnp.float32)

    b0 = p['edge_ln_b'][0]
    bias1 = p1['left_b'] + b0 * p1['edge_W'][:, 0]
    bias2 = p2['left_b'] + b0 * p2['edge_W'][:, 0]

    cons0, b1_tab = _node_mlp(
        constraint_features, p['cons_ln_g'], p['cons_ln_b'],
        p['cons_W1'].T, p['cons_b1'], p['cons_W2'].T, p['cons_b2'],
        [(p1['right_W'].T, zb)])
    var0, a1_tab, b2_tab = _node_mlp(
        variable_features, p['var_ln_g'], p['var_ln_b'],
        p['var_W1'].T, p['var_b1'], p['var_W2'].T, p['var_b2'],
        [(p1['left_W'].T, bias1), (p2['right_W'].T, zb)])

    j21 = _gather(a1_tab, b1_tab, ei[1], ei[0])
    msg1 = _msg(j21, p1['fin_g'], p1['fin_b'], p1['fin_W'].T,
                p1['fin_Wb'])
    agg1p = _scatter(msg1, ei[0])
    agg1 = _assemble_agg(agg1p, n)
    cons1, a2_tab = _post(
        agg1, cons0, p1['post_g'], p1['post_b'], p1['out1_W'].T,
        p1['out1_b'], p1['out2_W'].T, p1['out2_b'],
        [(p2['left_W'].T, bias2)], head=None)

    j22 = _gather(a2_tab, b2_tab, ei[0], ei[1])
    msg2 = _msg(j22, p2['fin_g'], p2['fin_b'], p2['fin_W'].T,
                p2['fin_Wb'])
    agg2p = _scatter(msg2, ei[1])
    agg2 = _assemble_agg(agg2p, n)
    out = _post(
        agg2, var0, p2['post_g'], p2['post_b'], p2['out1_W'].T,
        p2['out1_b'], p2['out2_W'].T, p2['out2_b'], [],
        head=(p['out_W1'].T, p['out_b1'], p['out_W2'].T))
    return out[:, 0]
